# baseline (speedup 1.0000x reference)
import functools

import numpy as np
import jax
import jax.numpy as jnp
from jax import lax
from jax.experimental import pallas as pl
from jax.experimental.pallas import tpu as pltpu
from jax.experimental.pallas import tpu_sc as plsc

_B = 4
_L = 2048
_D = 768
_NC = 2
_NS = 16
_NW = _NC * _NS
_PW = _L // _NW
_NB = 2
_DW = _D // 32


def _pe_packed() -> np.ndarray:
    import ml_dtypes

    pos = np.arange(_L, dtype=np.float32).reshape(-1, 1)
    exponent = np.arange(0, _D, 2, dtype=np.float32).reshape(1, -1) / np.float32(_D)
    X = (pos / np.power(np.float32(10000.0), exponent)).astype(np.float32)
    pe = np.zeros((_L, _D), dtype=np.float32)
    pe[:, 0::2] = np.sin(X)
    pe[:, 1::2] = np.cos(X)
    ub = pe.astype(ml_dtypes.bfloat16).view(np.uint16).astype(np.uint32)
    ub = ub.reshape(_L, _DW, 2, 16)
    packed = ub[:, :, 0, :] | (ub[:, :, 1, :] << 16)
    return packed.reshape(_L, _DW * 16).view(np.int32)


_MESH = plsc.VectorSubcoreMesh(core_axis_name="c", subcore_axis_name="s")


@functools.partial(
    pl.kernel,
    mesh=_MESH,
    out_type=jax.ShapeDtypeStruct((_B * _L, _D), jnp.float32),
    compiler_params=pltpu.CompilerParams(needs_layout_passes=False),
    scratch_types=[
        pltpu.VMEM((_B, _PW), jnp.int32),
        pltpu.VMEM((_PW, _DW * 16), jnp.int32),
        pltpu.VMEM((_NB, _PW, _D), jnp.float32),
        pltpu.SemaphoreType.DMA,
        pltpu.SemaphoreType.DMA,
        pltpu.SemaphoreType.DMA,
        pltpu.SemaphoreType.DMA,
        pltpu.SemaphoreType.DMA,
    ],
)
def _emb_kernel(x_hbm, table_hbm, pe_hbm, out_hbm, idx_v, pe_v, rows_v,
                gs0, gs1, ws0, ws1, psem):
    gsem = (gs0, gs1)
    wsem = (ws0, ws1)
    wid = lax.axis_index("s") * _NC + lax.axis_index("c")
    pos0 = wid * _PW

    pltpu.sync_copy(x_hbm.at[0, pl.ds(pos0, _PW)], idx_v.at[0])

    def start_gather(b):
        buf = b % _NB
        return pltpu.async_copy(
            table_hbm.at[idx_v.at[b]], rows_v.at[buf], gsem[buf])

    gh = {0: start_gather(0)}
    pe_h = pltpu.async_copy(pe_hbm.at[pl.ds(pos0, _PW)], pe_v, psem)
    for b in range(1, _B):
        pltpu.sync_copy(x_hbm.at[b, pl.ds(pos0, _PW)], idx_v.at[b])

    wh = {}
    for b in range(_B):
        nb = b + 1
        if nb < _B:
            if nb >= _NB:
                wh.pop(nb - _NB).wait()
            gh[nb] = start_gather(nb)
        gh.pop(b).wait()
        if b == 0:
            pe_h.wait()

        buf = b % _NB

        @plsc.parallel_loop(0, _PW, 1, unroll=8)
        def _add_row(r):
            for j in range(_DW):
                w = pe_v[r, pl.ds(j * 16, 16)]
                lo, hi = plsc.unpack(
                    plsc.bitcast(w, jnp.bfloat16),
                    format=plsc.PackFormat.INTERLEAVED,
                    preferred_element_type=jnp.float32)
                sl_lo = pl.ds(j * 32, 16)
                sl_hi = pl.ds(j * 32 + 16, 16)
                rows_v[buf, r, sl_lo] = rows_v[buf, r, sl_lo] + lo
                rows_v[buf, r, sl_hi] = rows_v[buf, r, sl_hi] + hi

        wh[b] = pltpu.async_copy(
            rows_v.at[buf], out_hbm.at[pl.ds(b * _L + pos0, _PW)], wsem[buf])

    for b in sorted(wh):
        wh.pop(b).wait()


_PE_CONST = _pe_packed()


def kernel(x, table):
    pe = jnp.asarray(_PE_CONST)
    out = _emb_kernel(x.astype(jnp.int32), table, pe)
    return out.reshape(_B, _L, _D)


# speedup vs baseline: 1.1046x; 1.1046x over previous
import functools

import numpy as np
import jax
import jax.numpy as jnp
from jax import lax
from jax.experimental import pallas as pl
from jax.experimental.pallas import tpu as pltpu
from jax.experimental.pallas import tpu_sc as plsc

_B = 4
_L = 2048
_D = 768
_NC = 2
_NS = 16
_NW = _NC * _NS
_PW = _L // _NW
_NB = 2
_DW = _D // 32


def _pe_packed() -> np.ndarray:
    import ml_dtypes

    pos = np.arange(_L, dtype=np.float32).reshape(-1, 1)
    exponent = np.arange(0, _D, 2, dtype=np.float32).reshape(1, -1) / np.float32(_D)
    X = (pos / np.power(np.float32(10000.0), exponent)).astype(np.float32)
    pe = np.zeros((_L, _D), dtype=np.float32)
    pe[:, 0::2] = np.sin(X)
    pe[:, 1::2] = np.cos(X)
    ub = pe.astype(ml_dtypes.bfloat16).view(np.uint16).astype(np.uint32)
    ub = ub.reshape(_L, _DW, 2, 16)
    packed = ub[:, :, 0, :] | (ub[:, :, 1, :] << 16)
    return packed.reshape(_L, _DW * 16).view(np.int32)


_MESH = plsc.VectorSubcoreMesh(core_axis_name="c", subcore_axis_name="s")


@functools.partial(
    pl.kernel,
    mesh=_MESH,
    out_type=jax.ShapeDtypeStruct((_B * _L, _D), jnp.float32),
    compiler_params=pltpu.CompilerParams(needs_layout_passes=False),
    scratch_types=[
        pltpu.VMEM((_B, _PW), jnp.int32),
        pltpu.VMEM((_PW, _DW * 16), jnp.int32),
        pltpu.VMEM((_NB, _PW, _D), jnp.float32),
        pltpu.SemaphoreType.DMA,
        pltpu.SemaphoreType.DMA,
        pltpu.SemaphoreType.DMA,
        pltpu.SemaphoreType.DMA,
        pltpu.SemaphoreType.DMA,
    ],
)
def _emb_kernel(x_hbm, table_hbm, pe_hbm, out_hbm, idx_v, pe_v, rows_v,
                gs0, gs1, ws0, ws1, psem):
    gsem = (gs0, gs1)
    wsem = (ws0, ws1)
    wid = lax.axis_index("s") * _NC + lax.axis_index("c")
    pos0 = wid * _PW

    pltpu.sync_copy(x_hbm.at[0, pl.ds(pos0, _PW)], idx_v.at[0])

    def start_gather(b):
        buf = b % _NB
        return pltpu.async_copy(
            table_hbm.at[idx_v.at[b]], rows_v.at[buf], gsem[buf])

    gh = {0: start_gather(0)}
    pe_h = pltpu.async_copy(pe_hbm.at[pl.ds(pos0, _PW)], pe_v, psem)
    for b in range(1, _B):
        pltpu.sync_copy(x_hbm.at[b, pl.ds(pos0, _PW)], idx_v.at[b])

    wh = {}
    for b in range(_B):
        nb = b + 1
        if nb < _B:
            if nb >= _NB:
                wh.pop(nb - _NB).wait()
            gh[nb] = start_gather(nb)
        gh.pop(b).wait()
        if b == 0:
            pe_h.wait()

        buf = b % _NB

        @plsc.parallel_loop(0, _PW, 1, unroll=4)
        def _add_row(r):
            for j in range(_DW):
                w = pe_v[r, pl.ds(j * 16, 16)]
                lo, hi = plsc.unpack(
                    plsc.bitcast(w, jnp.bfloat16),
                    format=plsc.PackFormat.INTERLEAVED,
                    preferred_element_type=jnp.float32)
                sl_lo = pl.ds(j * 32, 16)
                sl_hi = pl.ds(j * 32 + 16, 16)
                rows_v[buf, r, sl_lo] = rows_v[buf, r, sl_lo] + lo
                rows_v[buf, r, sl_hi] = rows_v[buf, r, sl_hi] + hi

        wh[b] = pltpu.async_copy(
            rows_v.at[buf], out_hbm.at[pl.ds(b * _L + pos0, _PW)], wsem[buf])

    for b in sorted(wh):
        wh.pop(b).wait()


_PE_CONST = _pe_packed()


def kernel(x, table):
    pe = jnp.asarray(_PE_CONST)
    out = _emb_kernel(x.astype(jnp.int32), table, pe)
    return out.reshape(_B, _L, _D)
